# Initial kernel scaffold; baseline (speedup 1.0000x reference)
import functools

import jax
import jax.numpy as jnp
from jax import lax
from jax.experimental import pallas as pl
from jax.experimental.pallas import tpu as pltpu
from jax.experimental.pallas import tpu_sc as plsc

E = 320000
D = 128
NC, NS, L = 2, 16, 16
NW = NC * NS
E_W = E // NW
C = 200
N_CHUNK = E_W // C

_mesh = plsc.VectorSubcoreMesh(core_axis_name="c", subcore_axis_name="s")


@functools.partial(
    pl.kernel,
    mesh=_mesh,
    out_type=jax.ShapeDtypeStruct((E,), jnp.float32),
    scratch_types=[
        pltpu.VMEM((C,), jnp.int32),
        pltpu.VMEM((C,), jnp.int32),
        pltpu.VMEM((C, D), jnp.float32),
        pltpu.VMEM((C, D), jnp.float32),
        pltpu.VMEM((C,), jnp.float32),
        pltpu.SemaphoreType.DMA,
    ],
)
def _edge_dot(drug, prot, idx0, idx1, out, idx0_v, idx1_v, r0, r1, o_v, sem):
    wid = lax.axis_index("s") * NC + lax.axis_index("c")
    base = wid * E_W

    def chunk_body(c, carry):
        off = base + c * C
        pltpu.sync_copy(idx0.at[pl.ds(off, C)], idx0_v)
        pltpu.sync_copy(idx1.at[pl.ds(off, C)], idx1_v)
        cp0 = pltpu.async_copy(drug.at[idx0_v], r0, sem)
        cp1 = pltpu.async_copy(prot.at[idx1_v], r1, sem)
        cp0.wait()
        cp1.wait()

        def edge_body(e, carry2):
            acc = r0[e, pl.ds(0, L)] * r1[e, pl.ds(0, L)]
            for k in range(1, D // L):
                acc = acc + r0[e, pl.ds(k * L, L)] * r1[e, pl.ds(k * L, L)]
            o_v[e] = jnp.sum(acc)
            return carry2

        lax.fori_loop(0, C, edge_body, 0)
        pltpu.sync_copy(o_v, out.at[pl.ds(off, C)])
        return carry

    lax.fori_loop(0, N_CHUNK, chunk_body, 0)


def kernel(x_drug, x_prot, edge_label_index):
    idx = edge_label_index.astype(jnp.int32)
    return _edge_dot(x_drug, x_prot, idx[0], idx[1])


# speedup vs baseline: 4.0413x; 4.0413x over previous
import functools

import jax
import jax.numpy as jnp
from jax import lax
from jax.experimental import pallas as pl
from jax.experimental.pallas import tpu as pltpu
from jax.experimental.pallas import tpu_sc as plsc

E = 320000
D = 128
NC, NS, L = 2, 16, 16
NW = NC * NS
E_W = E // NW
C = 200
N_CHUNK = E_W // C

_mesh = plsc.VectorSubcoreMesh(core_axis_name="c", subcore_axis_name="s")


@functools.partial(
    pl.kernel,
    mesh=_mesh,
    out_type=jax.ShapeDtypeStruct((E,), jnp.float32),
    scratch_types=[
        pltpu.VMEM((C,), jnp.int32),
        pltpu.VMEM((C,), jnp.int32),
        pltpu.VMEM((C, D), jnp.float32),
        pltpu.VMEM((C, D), jnp.float32),
        pltpu.VMEM((C + L,), jnp.float32),
        pltpu.SemaphoreType.DMA,
    ],
    compiler_params=pltpu.CompilerParams(needs_layout_passes=False),
)
def _edge_dot(drug, prot, idx0, idx1, out, idx0_v, idx1_v, r0, r1, o_v, sem):
    wid = lax.axis_index("s") * NC + lax.axis_index("c")
    base = wid * E_W
    last_lane = lax.iota(jnp.int32, L) == (L - 1)

    def chunk_body(c, carry):
        off = base + c * C
        pltpu.sync_copy(idx0.at[pl.ds(off, C)], idx0_v)
        pltpu.sync_copy(idx1.at[pl.ds(off, C)], idx1_v)
        cp0 = pltpu.async_copy(drug.at[idx0_v], r0, sem)
        cp1 = pltpu.async_copy(prot.at[idx1_v], r1, sem)
        cp0.wait()
        cp1.wait()

        def edge_body(e, carry2):
            acc = r0[e, pl.ds(0, L)] * r1[e, pl.ds(0, L)]
            for k in range(1, D // L):
                acc = acc + r0[e, pl.ds(k * L, L)] * r1[e, pl.ds(k * L, L)]
            cs = plsc.cumsum(acc)
            plsc.store_compressed(o_v.at[pl.ds(e, L)], cs, mask=last_lane)
            return carry2

        lax.fori_loop(0, C, edge_body, 0)
        pltpu.sync_copy(o_v.at[pl.ds(0, C)], out.at[pl.ds(off, C)])
        return carry

    lax.fori_loop(0, N_CHUNK, chunk_body, 0)


def kernel(x_drug, x_prot, edge_label_index):
    idx = edge_label_index.astype(jnp.int32)
    return _edge_dot(x_drug, x_prot, idx[0], idx[1])
